# baseline (speedup 1.0000x reference)
import jax
import jax.numpy as jnp
from jax.experimental import pallas as pl
from jax.experimental.pallas import tpu as pltpu

_N = 4096
_IN = 256
_E = 64
_H = 8
_ALPHA = 0.2
_BR = 512
_LOG2E = 1.4426950408889634


def _gat_kernel(x_ref, w0, w1, w2, w3_, w4, w5, w6, w7,
                a0, a1_, a2_, a3, a4, a5, a6, a7, adj_ref, out_ref,
                whg_s, c1_s, c2_s, e2a_s, e2b_s):
    i = pl.program_id(0)

    @pl.when(i == 0)
    def _proj():
        wcat = jnp.concatenate(
            [w0[...], w1[...], w2[...], w3_[...],
             w4[...], w5[...], w6[...], w7[...]], axis=1)
        a_rows = jnp.concatenate(
            [a0[...].reshape(1, 2 * _E), a1_[...].reshape(1, 2 * _E),
             a2_[...].reshape(1, 2 * _E), a3[...].reshape(1, 2 * _E),
             a4[...].reshape(1, 2 * _E), a5[...].reshape(1, 2 * _E),
             a6[...].reshape(1, 2 * _E), a7[...].reshape(1, 2 * _E)],
            axis=0)
        wgp = jnp.concatenate(
            [wcat.reshape(_IN, _H, _E),
             jnp.zeros((_IN, _H, 128 - _E), jnp.float32)],
            axis=2).reshape(_IN, _H * 128)
        lane = jax.lax.broadcasted_iota(jnp.int32, (1, _H * 128), 1)
        ones_pat = jnp.where(lane % 128 == _E, 1.0, 0.0)
        whg_s[...] = (jnp.dot(x_ref[...], wgp,
                              preferred_element_type=jnp.float32)
                      + ones_pat).astype(jnp.bfloat16)
        w3 = wcat.reshape(_IN, _H, _E)
        a1 = a_rows[:, :_E]
        a2 = a_rows[:, _E:]
        u1 = _LOG2E * jnp.sum(w3 * a1[None, :, :], axis=2)
        u2 = _LOG2E * jnp.sum(w3 * a2[None, :, :], axis=2)
        e1 = jnp.dot(x_ref[...], u1, preferred_element_type=jnp.float32)
        e2c = jnp.dot(x_ref[...], u2, preferred_element_type=jnp.float32)
        m2 = jnp.max(e2c, axis=0, keepdims=True)
        w = e1 + m2
        mh = jnp.maximum(w, _ALPHA * w)
        c1_s[...] = e1 - mh
        c2_s[...] = _ALPHA * e1 - mh
        e2a = jax.lax.dot_general(
            u2, x_ref[...], dimension_numbers=(((0,), (1,)), ((), ())),
            preferred_element_type=jnp.float32)
        e2a_s[...] = e2a
        e2b_s[...] = _ALPHA * e2a

    adjb = adj_ref[...]
    r0 = i * _BR
    for h in range(_H):
        c1 = c1_s[pl.ds(r0, _BR), h][:, None]
        c2 = c2_s[pl.ds(r0, _BR), h][:, None]
        t = jnp.maximum(c1 + e2a_s[h, :][None, :],
                        c2 + e2b_s[h, :][None, :])
        p = jnp.exp2(t) * adjb
        res = jnp.dot(p, whg_s[:, h * 128:(h + 1) * 128],
                      preferred_element_type=jnp.float32)
        s = res[:, _E:_E + 1]
        hp = res[:, :_E] * (1.0 / jnp.maximum(s, 1e-30))
        out_ref[:, h * _E:(h + 1) * _E] = jnp.where(hp > 0.0, hp, jnp.exp(hp) - 1.0)


@jax.jit
def kernel(x, adj, W0, a0, W1, a1, W2, a2, W3, a3, W4, a4, W5, a5, W6, a6, W7, a7):
    nblk = _N // _BR
    w_spec = pl.BlockSpec((_IN, _E), lambda i: (0, 0))
    a_spec = pl.BlockSpec((2 * _E, 1), lambda i: (0, 0))
    out = pl.pallas_call(
        _gat_kernel,
        grid=(nblk,),
        in_specs=[
            pl.BlockSpec((_N, _IN), lambda i: (0, 0)),
            w_spec, w_spec, w_spec, w_spec, w_spec, w_spec, w_spec, w_spec,
            a_spec, a_spec, a_spec, a_spec, a_spec, a_spec, a_spec, a_spec,
            pl.BlockSpec((_BR, _N), lambda i: (i, 0)),
        ],
        out_specs=pl.BlockSpec((_BR, _H * _E), lambda i: (i, 0)),
        out_shape=jax.ShapeDtypeStruct((_N, _H * _E), jnp.float32),
        scratch_shapes=[
            pltpu.VMEM((_N, _H * 128), jnp.bfloat16),
            pltpu.VMEM((_N, _H), jnp.float32),
            pltpu.VMEM((_N, _H), jnp.float32),
            pltpu.VMEM((_H, _N), jnp.float32),
            pltpu.VMEM((_H, _N), jnp.float32),
        ],
        compiler_params=pltpu.CompilerParams(
            dimension_semantics=("arbitrary",),
        ),
    )(x, W0, W1, W2, W3, W4, W5, W6, W7,
      a0, a1, a2, a3, a4, a5, a6, a7, adj)
    return out


# speedup vs baseline: 1.0767x; 1.0767x over previous
import jax
import jax.numpy as jnp
from jax.experimental import pallas as pl
from jax.experimental.pallas import tpu as pltpu

_N = 4096
_IN = 256
_E = 64
_H = 8
_ALPHA = 0.2
_BR = 256
_LOG2E = 1.4426950408889634


def _gat_kernel(x_ref, w_ref, a_ref, adj_ref, out_ref,
                whg_s, c1_s, c2_s, e2a_s, e2b_s):
    i = pl.program_id(0)

    @pl.when(i == 0)
    def _proj():
        wcat = w_ref[...]
        a_rows = a_ref[...]
        wgp = jnp.concatenate(
            [wcat.reshape(_IN, _H, _E),
             jnp.zeros((_IN, _H, 128 - _E), jnp.float32)],
            axis=2).reshape(_IN, _H * 128)
        lane = jax.lax.broadcasted_iota(jnp.int32, (1, _H * 128), 1)
        ones_pat = jnp.where(lane % 128 == _E, 1.0, 0.0)
        whg_s[...] = (jnp.dot(x_ref[...], wgp,
                              preferred_element_type=jnp.float32)
                      + ones_pat).astype(jnp.bfloat16)
        w3 = wcat.reshape(_IN, _H, _E)
        a1 = a_rows[:, :_E]
        a2 = a_rows[:, _E:]
        u1 = _LOG2E * jnp.sum(w3 * a1[None, :, :], axis=2)
        u2 = _LOG2E * jnp.sum(w3 * a2[None, :, :], axis=2)
        e1 = jnp.dot(x_ref[...], u1, preferred_element_type=jnp.float32)
        e2c = jnp.dot(x_ref[...], u2, preferred_element_type=jnp.float32)
        m2 = jnp.max(e2c, axis=0, keepdims=True)
        w = e1 + m2
        mh = jnp.maximum(w, _ALPHA * w)
        c1_s[...] = e1 - mh
        c2_s[...] = _ALPHA * e1 - mh
        e2a = jax.lax.dot_general(
            u2, x_ref[...], dimension_numbers=(((0,), (1,)), ((), ())),
            preferred_element_type=jnp.float32)
        e2a_s[...] = e2a
        e2b_s[...] = _ALPHA * e2a

    adjb = adj_ref[...]
    r0 = i * _BR
    for h in range(_H):
        c1 = c1_s[pl.ds(r0, _BR), h][:, None]
        c2 = c2_s[pl.ds(r0, _BR), h][:, None]
        t = jnp.maximum(c1 + e2a_s[h, :][None, :],
                        c2 + e2b_s[h, :][None, :])
        p = jnp.exp2(t) * adjb
        res = jnp.dot(p, whg_s[:, h * 128:(h + 1) * 128],
                      preferred_element_type=jnp.float32)
        s = res[:, _E:_E + 1]
        hp = res[:, :_E] * (1.0 / jnp.maximum(s, 1e-30))
        out_ref[:, h * _E:(h + 1) * _E] = jnp.where(hp > 0.0, hp, jnp.exp(hp) - 1.0)


@jax.jit
def kernel(x, adj, W0, a0, W1, a1, W2, a2, W3, a3, W4, a4, W5, a5, W6, a6, W7, a7):
    Wcat = jnp.concatenate([W0, W1, W2, W3, W4, W5, W6, W7], axis=1)
    acat = jnp.stack([a0, a1, a2, a3, a4, a5, a6, a7], axis=0)[..., 0]

    nblk = _N // _BR
    out = pl.pallas_call(
        _gat_kernel,
        grid=(nblk,),
        in_specs=[
            pl.BlockSpec((_N, _IN), lambda i: (0, 0)),
            pl.BlockSpec((_IN, _H * _E), lambda i: (0, 0)),
            pl.BlockSpec((_H, 2 * _E), lambda i: (0, 0)),
            pl.BlockSpec((_BR, _N), lambda i: (i, 0)),
        ],
        out_specs=pl.BlockSpec((_BR, _H * _E), lambda i: (i, 0)),
        out_shape=jax.ShapeDtypeStruct((_N, _H * _E), jnp.float32),
        scratch_shapes=[
            pltpu.VMEM((_N, _H * 128), jnp.bfloat16),
            pltpu.VMEM((_N, _H), jnp.float32),
            pltpu.VMEM((_N, _H), jnp.float32),
            pltpu.VMEM((_H, _N), jnp.float32),
            pltpu.VMEM((_H, _N), jnp.float32),
        ],
        compiler_params=pltpu.CompilerParams(
            dimension_semantics=("arbitrary",),
        ),
    )(x, Wcat, acat, adj)
    return out


# speedup vs baseline: 1.1234x; 1.0434x over previous
import jax
import jax.numpy as jnp
from jax.experimental import pallas as pl
from jax.experimental.pallas import tpu as pltpu

_N = 4096
_IN = 256
_E = 64
_H = 8
_ALPHA = 0.2
_BR = 512
_LOG2E = 1.4426950408889634


def _gat_kernel(x_ref, w_ref, a_ref, adj_ref, out_ref,
                whg_s, c1_s, c2_s, e2a_s, e2b_s):
    i = pl.program_id(0)

    @pl.when(i == 0)
    def _proj():
        wcat = w_ref[...]
        a_rows = a_ref[...]
        wgp = jnp.concatenate(
            [wcat.reshape(_IN, _H, _E),
             jnp.zeros((_IN, _H, 128 - _E), jnp.float32)],
            axis=2).reshape(_IN, _H * 128)
        lane = jax.lax.broadcasted_iota(jnp.int32, (1, _H * 128), 1)
        ones_pat = jnp.where(lane % 128 == _E, 1.0, 0.0)
        whg_s[...] = (jnp.dot(x_ref[...], wgp,
                              preferred_element_type=jnp.float32)
                      + ones_pat).astype(jnp.bfloat16)
        w3 = wcat.reshape(_IN, _H, _E)
        a1 = a_rows[:, :_E]
        a2 = a_rows[:, _E:]
        u1 = _LOG2E * jnp.sum(w3 * a1[None, :, :], axis=2)
        u2 = _LOG2E * jnp.sum(w3 * a2[None, :, :], axis=2)
        e1 = jnp.dot(x_ref[...], u1, preferred_element_type=jnp.float32)
        e2c = jnp.dot(x_ref[...], u2, preferred_element_type=jnp.float32)
        m2 = jnp.max(e2c, axis=0, keepdims=True)
        w = e1 + m2
        mh = jnp.maximum(w, _ALPHA * w)
        c1_s[...] = e1 - mh
        c2_s[...] = _ALPHA * e1 - mh
        e2a = jax.lax.dot_general(
            u2, x_ref[...], dimension_numbers=(((0,), (1,)), ((), ())),
            preferred_element_type=jnp.float32)
        e2a_s[...] = e2a
        e2b_s[...] = _ALPHA * e2a

    adjb = adj_ref[...]
    r0 = i * _BR
    for h in range(_H):
        c1 = c1_s[pl.ds(r0, _BR), h][:, None]
        c2 = c2_s[pl.ds(r0, _BR), h][:, None]
        t = jnp.maximum(c1 + e2a_s[h, :][None, :],
                        c2 + e2b_s[h, :][None, :])
        p = jnp.exp2(t) * adjb
        res = jnp.dot(p, whg_s[:, h * 128:(h + 1) * 128],
                      preferred_element_type=jnp.float32)
        s = res[:, _E:_E + 1]
        hp = res[:, :_E] * (1.0 / jnp.maximum(s, 1e-30))
        out_ref[:, h * _E:(h + 1) * _E] = jnp.where(hp > 0.0, hp, jnp.exp(hp) - 1.0)


@jax.jit
def kernel(x, adj, W0, a0, W1, a1, W2, a2, W3, a3, W4, a4, W5, a5, W6, a6, W7, a7):
    Wcat = jnp.concatenate([W0, W1, W2, W3, W4, W5, W6, W7], axis=1)
    acat = jnp.stack([a0, a1, a2, a3, a4, a5, a6, a7], axis=0)[..., 0]

    nblk = _N // _BR
    out = pl.pallas_call(
        _gat_kernel,
        grid=(nblk,),
        in_specs=[
            pl.BlockSpec((_N, _IN), lambda i: (0, 0)),
            pl.BlockSpec((_IN, _H * _E), lambda i: (0, 0)),
            pl.BlockSpec((_H, 2 * _E), lambda i: (0, 0)),
            pl.BlockSpec((_BR, _N), lambda i: (i, 0)),
        ],
        out_specs=pl.BlockSpec((_BR, _H * _E), lambda i: (i, 0)),
        out_shape=jax.ShapeDtypeStruct((_N, _H * _E), jnp.float32),
        scratch_shapes=[
            pltpu.VMEM((_N, _H * 128), jnp.bfloat16),
            pltpu.VMEM((_N, _H), jnp.float32),
            pltpu.VMEM((_N, _H), jnp.float32),
            pltpu.VMEM((_H, _N), jnp.float32),
            pltpu.VMEM((_H, _N), jnp.float32),
        ],
        compiler_params=pltpu.CompilerParams(
            dimension_semantics=("arbitrary",),
        ),
    )(x, Wcat, acat, adj)
    return out
